# baseline (speedup 1.0000x reference)
import functools

import jax
import jax.numpy as jnp
from jax import lax
from jax.experimental import pallas as pl
from jax.experimental.pallas import tpu as pltpu
from jax.experimental.pallas import tpu_sc as plsc

_B = 8
_N = 2048
_L = 16
_NC = 2
_NS = 16
_NCH = _N // _L
_UNROLL = 16
_RB = 256


def _cost_tc_kernel(p_ref, t_ref, o_ref):
    p = p_ref[0]
    t = t_ref[0]
    px, py, pz = p[:, 0:1], p[:, 1:2], p[:, 2:3]
    tx, ty, tz = t[0:1, :], t[1:2, :], t[2:3, :]
    dx = px - tx
    dy = py - ty
    dz = pz - tz
    d2 = dx * dx + dy * dy + dz * dz
    o_ref[0] = jnp.sqrt(jnp.maximum(d2, 1e-12))


_GATHER_DNUMS = lax.GatherDimensionNumbers(
    offset_dims=(), collapsed_slice_dims=(0,), start_index_map=(0,))


def _shuffle(v, perm):
    return lax.gather(v, perm[:, None], _GATHER_DNUMS, (1,),
                      mode=lax.GatherScatterMode.PROMISE_IN_BOUNDS)


def _allmin(v, lane):
    for k in (1, 2, 4, 8):
        v = jnp.minimum(v, _shuffle(v, lane ^ k))
    return v


_RPB = 16
_NBLK = _N // _RPB
_USM = 8


def _greedy_sc_body(cost_hbm, out_hbm, blk0, blk1, cand_v, pos_v, tot_v,
                    sem0, sem1):
    cid = lax.axis_index("c")
    sid = lax.axis_index("s")
    wid = sid * _NC + cid

    @pl.when(wid < _B)
    def _():
        lane = lax.iota(jnp.int32, _L)

        def initc(c, carry):
            cand_v[pl.ds(c * _L, _L)] = lane + c * _L
            pos_v[pl.ds(c * _L, _L)] = lane + c * _L
            return carry

        lax.fori_loop(0, _NCH, initc, 0, unroll=8)

        big_i = jnp.full((_L,), jnp.int32(2 ** 30))
        inf_f = jnp.full((_L,), jnp.float32(jnp.inf))

        def merge(minv, minj, v, jc):
            upd = (v < minv) | ((v == minv) & (jc < minj))
            return jnp.where(upd, v, minv), jnp.where(upd, jc, minj)

        def do_row(blk, r, i, total):
            n = _N - i
            rvec = jnp.full((_L,), r)
            fc = n // _L
            rem = n - fc * _L

            def scan_chunk(off, carry):
                jc = cand_v[pl.ds(off, _L)]
                v = plsc.load_gather(blk, [rvec, jc])
                return merge(*carry, v, jc)

            def blocked(carry):
                nb = (fc + _USM - 1) // _USM

                def blk_body(bi, c):
                    off0 = jnp.minimum(bi * _USM, fc - _USM) * _L
                    for u in range(_USM):
                        c = scan_chunk(off0 + u * _L, c)
                    return c

                return lax.fori_loop(0, nb, blk_body, carry)

            def small(carry):
                return lax.fori_loop(
                    0, fc, lambda ci, c: scan_chunk(ci * _L, c), carry)

            minv, minj = lax.cond(n >= _USM * _L, blocked, small,
                                  (inf_f, big_i))

            def tail(carry):
                off = jnp.maximum(n - _L, 0)
                jc = cand_v[pl.ds(off, _L)]
                v = plsc.load_gather(blk, [rvec, jc])
                v = jnp.where(lane < (n - off), v, inf_f)
                return merge(*carry, v, jc)

            minv, minj = lax.cond(rem > 0, tail, lambda c: c, (minv, minj))

            for k in (1, 2, 4, 8):
                ov = _shuffle(minv, lane ^ k)
                oj = _shuffle(minj, lane ^ k)
                minv, minj = merge(minv, minj, ov, oj)

            j = minj
            last = jnp.full((_L,), n - 1)
            jl = plsc.load_gather(cand_v, [last])
            p = plsc.load_gather(pos_v, [j])
            m0 = lane == 0
            plsc.store_scatter(cand_v, [p], jl, mask=m0)
            plsc.store_scatter(pos_v, [jl], p, mask=m0)
            return total + minv

        def fetch(bidx, blk, sem):
            pltpu.make_async_copy(
                cost_hbm.at[wid, pl.ds(bidx * _RPB, _RPB)], blk, sem).start()

        def drain(blk, sem):
            pltpu.make_async_copy(
                cost_hbm.at[wid, pl.ds(0, _RPB)], blk, sem).wait()

        fetch(0, blk0, sem0)
        fetch(1, blk1, sem1)

        def pair(pb, total):
            i0 = 2 * pb * _RPB
            drain(blk0, sem0)
            total = lax.fori_loop(
                0, _RPB, lambda r, t: do_row(blk0, r, i0 + r, t), total)

            @pl.when(2 * pb + 2 < _NBLK)
            def _():
                fetch(2 * pb + 2, blk0, sem0)

            drain(blk1, sem1)
            total = lax.fori_loop(
                0, _RPB, lambda r, t: do_row(blk1, r, i0 + _RPB + r, t),
                total)

            @pl.when(2 * pb + 3 < _NBLK)
            def _():
                fetch(2 * pb + 3, blk1, sem1)

            return total

        total = lax.fori_loop(0, _NBLK // 2, pair,
                              jnp.zeros((_L,), jnp.float32))
        tot_v[...] = total
        pltpu.sync_copy(tot_v, out_hbm.at[wid])


def _build_cost(pred, target_t):
    return pl.pallas_call(
        _cost_tc_kernel,
        grid=(_B, _N // _RB),
        in_specs=[
            pl.BlockSpec((1, _RB, 3), lambda b, r: (b, r, 0)),
            pl.BlockSpec((1, 3, _N), lambda b, r: (b, 0, 0)),
        ],
        out_specs=pl.BlockSpec((1, _RB, _N), lambda b, r: (b, r, 0)),
        out_shape=jax.ShapeDtypeStruct((_B, _N, _N), jnp.float32),
    )(pred, target_t)


def _run_greedy(cost):
    mesh = plsc.VectorSubcoreMesh(core_axis_name="c", subcore_axis_name="s",
                                  num_cores=_NC, num_subcores=_NS)
    return pl.kernel(
        _greedy_sc_body,
        out_type=jax.ShapeDtypeStruct((_B, _L), jnp.float32),
        mesh=mesh,
        scratch_types=[
            pltpu.VMEM((_RPB, _N), jnp.float32),
            pltpu.VMEM((_RPB, _N), jnp.float32),
            pltpu.VMEM((_N,), jnp.int32),
            pltpu.VMEM((_N,), jnp.int32),
            pltpu.VMEM((_L,), jnp.float32),
            pltpu.SemaphoreType.DMA,
            pltpu.SemaphoreType.DMA,
        ],
        compiler_params=pltpu.CompilerParams(needs_layout_passes=False),
    )(cost)


@jax.jit
def kernel(pred, target):
    target_t = jnp.transpose(target, (0, 2, 1))
    cost = _build_cost(pred, target_t)
    totals = _run_greedy(cost)
    return jnp.mean(totals[:, 0] / _N)


# speedup vs baseline: 1.4215x; 1.4215x over previous
import functools

import jax
import jax.numpy as jnp
from jax import lax
from jax.experimental import pallas as pl
from jax.experimental.pallas import tpu as pltpu
from jax.experimental.pallas import tpu_sc as plsc

_B = 8
_N = 2048
_L = 16
_NC = 2
_NS = 16
_NCH = _N // _L
_UNROLL = 16
_RB = 256


def _cost_tc_kernel(p_ref, t_ref, o_ref):
    p = p_ref[0]
    t = t_ref[0]
    px, py, pz = p[:, 0:1], p[:, 1:2], p[:, 2:3]
    tx, ty, tz = t[0:1, :], t[1:2, :], t[2:3, :]
    dx = px - tx
    dy = py - ty
    dz = pz - tz
    d2 = dx * dx + dy * dy + dz * dz
    o_ref[0] = jnp.maximum(d2, 1e-12)


_GATHER_DNUMS = lax.GatherDimensionNumbers(
    offset_dims=(), collapsed_slice_dims=(0,), start_index_map=(0,))


def _shuffle(v, perm):
    return lax.gather(v, perm[:, None], _GATHER_DNUMS, (1,),
                      mode=lax.GatherScatterMode.PROMISE_IN_BOUNDS)


def _allmin(v, lane):
    for k in (1, 2, 4, 8):
        v = jnp.minimum(v, _shuffle(v, lane ^ k))
    return v


_RPB = 16
_NBLK = _N // _RPB


def _mean_sqrt_tc_kernel(x_ref, o_ref):
    s = jnp.sum(jnp.sqrt(x_ref[...])) / (_B * _N)
    o_ref[...] = jnp.full((1, 1), s)


def _greedy_sc_body(cost_hbm, out_hbm, blk0, blk1, pen_v, sel_v, sem0, sem1):
    cid = lax.axis_index("c")
    sid = lax.axis_index("s")
    wid = sid * _NC + cid

    @pl.when(wid < _B)
    def _():
        zero = jnp.zeros((_L,), jnp.float32)

        def initc(c, carry):
            pen_v[pl.ds(c * _L, _L)] = zero
            return carry

        lax.fori_loop(0, _NCH, initc, 0, unroll=8)

        lane = lax.iota(jnp.int32, _L)
        big_i = jnp.full((_L,), jnp.int32(2 ** 30))
        inf_f = jnp.full((_L,), jnp.float32(jnp.inf))

        def do_row(blk, r, i, total):
            def chunks(cc, carry):
                minv, mini = carry
                for u in range(_UNROLL):
                    c = cc * _UNROLL + u
                    v = blk[r, pl.ds(c * _L, _L)] + pen_v[pl.ds(c * _L, _L)]
                    idx = lane + c * _L
                    upd = v < minv
                    minv = jnp.where(upd, v, minv)
                    mini = jnp.where(upd, idx, mini)
                return minv, mini

            minv, mini = lax.fori_loop(0, _NCH // _UNROLL, chunks,
                                       (inf_f, big_i))
            gmin = _allmin(minv, lane)
            cand = jnp.where(minv == gmin, mini, big_i)
            j = _allmin(cand, lane)
            m0 = lane == 0
            plsc.store_scatter(pen_v, [j], inf_f, mask=m0)
            plsc.store_scatter(sel_v, [jnp.full((_L,), i)], gmin, mask=m0)
            return total

        def fetch(bidx, blk, sem):
            pltpu.make_async_copy(
                cost_hbm.at[wid, pl.ds(bidx * _RPB, _RPB)], blk, sem).start()

        def drain(blk, sem):
            pltpu.make_async_copy(
                cost_hbm.at[wid, pl.ds(0, _RPB)], blk, sem).wait()

        fetch(0, blk0, sem0)
        fetch(1, blk1, sem1)

        def pair(pb, total):
            i0 = 2 * pb * _RPB
            drain(blk0, sem0)
            total = lax.fori_loop(
                0, _RPB, lambda r, t: do_row(blk0, r, i0 + r, t), total)

            @pl.when(2 * pb + 2 < _NBLK)
            def _():
                fetch(2 * pb + 2, blk0, sem0)

            drain(blk1, sem1)
            total = lax.fori_loop(
                0, _RPB, lambda r, t: do_row(blk1, r, i0 + _RPB + r, t),
                total)

            @pl.when(2 * pb + 3 < _NBLK)
            def _():
                fetch(2 * pb + 3, blk1, sem1)

            return total

        lax.fori_loop(0, _NBLK // 2, pair, 0)
        pltpu.sync_copy(sel_v, out_hbm.at[wid])


def _build_cost(pred, target_t):
    return pl.pallas_call(
        _cost_tc_kernel,
        grid=(_B, _N // _RB),
        in_specs=[
            pl.BlockSpec((1, _RB, 3), lambda b, r: (b, r, 0)),
            pl.BlockSpec((1, 3, _N), lambda b, r: (b, 0, 0)),
        ],
        out_specs=pl.BlockSpec((1, _RB, _N), lambda b, r: (b, r, 0)),
        out_shape=jax.ShapeDtypeStruct((_B, _N, _N), jnp.float32),
    )(pred, target_t)


def _run_greedy(cost):
    mesh = plsc.VectorSubcoreMesh(core_axis_name="c", subcore_axis_name="s",
                                  num_cores=_NC, num_subcores=_NS)
    return pl.kernel(
        _greedy_sc_body,
        out_type=jax.ShapeDtypeStruct((_B, _N), jnp.float32),
        mesh=mesh,
        scratch_types=[
            pltpu.VMEM((_RPB, _N), jnp.float32),
            pltpu.VMEM((_RPB, _N), jnp.float32),
            pltpu.VMEM((_N,), jnp.float32),
            pltpu.VMEM((_N,), jnp.float32),
            pltpu.SemaphoreType.DMA,
            pltpu.SemaphoreType.DMA,
        ],
        compiler_params=pltpu.CompilerParams(needs_layout_passes=False),
    )(cost)


def _finish(sel):
    return pl.pallas_call(
        _mean_sqrt_tc_kernel,
        out_shape=jax.ShapeDtypeStruct((1, 1), jnp.float32),
    )(sel)


@jax.jit
def kernel(pred, target):
    target_t = jnp.transpose(target, (0, 2, 1))
    cost = _build_cost(pred, target_t)
    sel = _run_greedy(cost)
    return _finish(sel)[0, 0]


# speedup vs baseline: 1.8183x; 1.2791x over previous
import functools

import jax
import jax.numpy as jnp
from jax import lax
from jax.experimental import pallas as pl
from jax.experimental.pallas import tpu as pltpu
from jax.experimental.pallas import tpu_sc as plsc

_B = 8
_N = 2048
_L = 16
_NC = 2
_NS = 16
_NCH = _N // _L
_UNROLL = 16
_RB = 256
_RPB = 16
_NBLK = _N // _RPB


def _cost_tc_kernel(p_ref, t_ref, o_ref):
    p = p_ref[0]
    t = t_ref[0]
    px, py, pz = p[:, 0:1], p[:, 1:2], p[:, 2:3]
    tx, ty, tz = t[0:1, :], t[1:2, :], t[2:3, :]
    dx = px - tx
    dy = py - ty
    dz = pz - tz
    d2 = dx * dx + dy * dy + dz * dz
    o_ref[0] = jnp.sqrt(jnp.maximum(d2, 1e-12))


_GATHER_DNUMS = lax.GatherDimensionNumbers(
    offset_dims=(), collapsed_slice_dims=(0,), start_index_map=(0,))


def _shuffle(v, perm):
    return lax.gather(v, perm[:, None], _GATHER_DNUMS, (1,),
                      mode=lax.GatherScatterMode.PROMISE_IN_BOUNDS)


def _allmin(v, lane):
    for k in (1, 2, 4, 8):
        v = jnp.minimum(v, _shuffle(v, lane ^ k))
    return v


def _greedy_sc_body(cost_hbm, out_hbm, blk0, blk1, pen_v, stage_i, tot_v,
                    inbox, sem0, sem1):
    cid = lax.axis_index("c")
    sid = lax.axis_index("s")
    active = sid < _B
    batch = cid * 4 + sid // 2
    role = sid & 1
    partner = sid ^ 1

    @pl.when(active)
    def _():
        inbox[0] = 0
        inbox[1] = 0

    plsc.subcore_barrier()

    @pl.when(active)
    def _():
        lane = lax.iota(jnp.int32, _L)
        zero = jnp.zeros((_L,), jnp.float32)

        def initc(c, carry):
            pen_v[pl.ds(c * _L, _L)] = zero
            return carry

        lax.fori_loop(0, _NCH, initc, 0, unroll=8)

        big_i = jnp.full((_L,), jnp.int32(2 ** 30))
        inf_f = jnp.full((_L,), jnp.float32(jnp.inf))

        def argmin_row(blk, r):
            def chunks(cc, carry):
                minv, mini = carry
                for u in range(_UNROLL):
                    c = cc * _UNROLL + u
                    v = blk[r, pl.ds(c * _L, _L)] + pen_v[pl.ds(c * _L, _L)]
                    idx = lane + c * _L
                    upd = v < minv
                    minv = jnp.where(upd, v, minv)
                    mini = jnp.where(upd, idx, mini)
                return minv, mini

            minv, mini = lax.fori_loop(0, _NCH // _UNROLL, chunks,
                                       (inf_f, big_i))
            gmin = _allmin(minv, lane)
            cand = jnp.where(minv == gmin, mini, big_i)
            jv = _allmin(cand, lane)
            return gmin, jnp.min(jv)

        def poll(slot):
            def cond(v):
                return v == 0

            def body(v):
                return plsc.fetch_and_add(inbox.at[slot], 0, subcore_id=sid)

            first = plsc.fetch_and_add(inbox.at[slot], 0, subcore_id=sid)
            val = lax.while_loop(cond, body, first)
            inbox[slot] = 0
            return val - 1

        def mask_col(js):
            plsc.store_scatter(pen_v, [jnp.full((_L,), js)], inf_f,
                               mask=(lane == 0))

        def do_row(blk, r, i, total):
            gmin, js = argmin_row(blk, r)

            def with_recv(args):
                gmin, js = args
                jprev = poll(((i - 1) >> 1) & 1)
                mask_col(jprev)
                return lax.cond(js == jprev,
                                lambda _: argmin_row(blk, r),
                                lambda a: a, (gmin, js))

            gmin, js = lax.cond(i > 0, with_recv, lambda a: a, (gmin, js))
            plsc.fetch_and_add(inbox.at[(i >> 1) & 1], js + 1,
                               subcore_id=partner)
            mask_col(js)
            return total + gmin

        def fetch(bidx, blk, sem):
            pltpu.make_async_copy(
                cost_hbm.at[batch, pl.ds(bidx * _RPB, _RPB)], blk,
                sem).start()

        def drain(blk, sem):
            pltpu.make_async_copy(
                cost_hbm.at[batch, pl.ds(0, _RPB)], blk, sem).wait()

        fetch(0, blk0, sem0)
        fetch(1, blk1, sem1)

        def rows(blk, base, total):
            def one(rl, t):
                r = 2 * rl + role
                return do_row(blk, r, base + r, t)

            return lax.fori_loop(0, _RPB // 2, one, total)

        def pair(pb, total):
            i0 = 2 * pb * _RPB
            drain(blk0, sem0)
            total = rows(blk0, i0, total)

            @pl.when(2 * pb + 2 < _NBLK)
            def _():
                fetch(2 * pb + 2, blk0, sem0)

            drain(blk1, sem1)
            total = rows(blk1, i0 + _RPB, total)

            @pl.when(2 * pb + 3 < _NBLK)
            def _():
                fetch(2 * pb + 3, blk1, sem1)

            return total

        total = lax.fori_loop(0, _NBLK // 2, pair,
                              jnp.zeros((_L,), jnp.float32))
        tot_v[...] = total
        pltpu.sync_copy(tot_v, out_hbm.at[cid * _B + sid])


def _build_cost(pred, target_t):
    return pl.pallas_call(
        _cost_tc_kernel,
        grid=(_B, _N // _RB),
        in_specs=[
            pl.BlockSpec((1, _RB, 3), lambda b, r: (b, r, 0)),
            pl.BlockSpec((1, 3, _N), lambda b, r: (b, 0, 0)),
        ],
        out_specs=pl.BlockSpec((1, _RB, _N), lambda b, r: (b, r, 0)),
        out_shape=jax.ShapeDtypeStruct((_B, _N, _N), jnp.float32),
    )(pred, target_t)


def _run_greedy(cost):
    mesh = plsc.VectorSubcoreMesh(core_axis_name="c", subcore_axis_name="s",
                                  num_cores=_NC, num_subcores=_NS)
    return pl.kernel(
        _greedy_sc_body,
        out_type=jax.ShapeDtypeStruct((2 * _B, _L), jnp.float32),
        mesh=mesh,
        scratch_types=[
            pltpu.VMEM((_RPB, _N), jnp.float32),
            pltpu.VMEM((_RPB, _N), jnp.float32),
            pltpu.VMEM((_N,), jnp.float32),
            pltpu.VMEM((_L,), jnp.int32),
            pltpu.VMEM((_L,), jnp.float32),
            pltpu.SMEM((2,), jnp.int32),
            pltpu.SemaphoreType.DMA,
            pltpu.SemaphoreType.DMA,
        ],
        compiler_params=pltpu.CompilerParams(needs_layout_passes=False),
    )(cost)


@jax.jit
def kernel(pred, target):
    target_t = jnp.transpose(target, (0, 2, 1))
    cost = _build_cost(pred, target_t)
    totals = _run_greedy(cost)
    return jnp.sum(totals[:, 0]) / (_B * _N)
